# baseline (speedup 1.0000x reference)
import functools

import jax
import jax.numpy as jnp
import numpy as np
from jax import lax
from jax.experimental import pallas as pl
from jax.experimental.pallas import tpu as pltpu
from jax.experimental.pallas import tpu_sc as plsc

IN_DIM = 2048
OUT_DIM = 8192
BATCH = 2048
TAU = 1.0

NC = 2
NS = 16
L = 16
NW = NC * NS
ROWS_PER_W = BATCH // NW
RG = 8

_M = np.zeros((16, 4), np.float32)
for _k, (_al, _be, _ga, _de) in {
    1: (0, 0, 0, 1), 2: (0, 1, 0, -1), 3: (0, 1, 0, 0), 4: (0, 0, 1, -1),
    5: (0, 0, 1, 0), 6: (0, 1, 1, -2), 7: (0, 1, 1, -1), 8: (1, -1, -1, 1),
    9: (1, -1, -1, 2), 10: (1, 0, -1, 0), 11: (1, 0, -1, 1), 12: (1, -1, 0, 0),
    13: (1, -1, 0, 1), 14: (1, 0, 0, -1), 15: (1, 0, 0, 0),
}.items():
    _M[_k] = [_al, _be, _ga, _de]
_MT = _M.T.copy()


def _coef_body(mt_ref, wt_ref, idx_ref, coef_ref, pidx_ref):
    w = wt_ref[...] * (1.0 / TAU)
    m = jnp.max(w, axis=0, keepdims=True)
    e = jnp.exp(w - m)
    p = e / jnp.sum(e, axis=0, keepdims=True)
    coef_ref[...] = jax.lax.dot_general(
        mt_ref[...], p, (((1,), (0,)), ((), ())),
        preferred_element_type=jnp.float32)
    pidx_ref[...] = idx_ref[0:1] | (idx_ref[1:2] << 16)


def _coefs(weight_t, indices):
    return pl.pallas_call(
        _coef_body,
        out_shape=(
            jax.ShapeDtypeStruct((4, OUT_DIM), jnp.float32),
            jax.ShapeDtypeStruct((1, OUT_DIM), jnp.int32),
        ),
    )(jnp.asarray(_MT), weight_t, indices)


def _sc_gate_kernel(x, pidx, coef):
    mesh = plsc.VectorSubcoreMesh(core_axis_name="c", subcore_axis_name="s")

    @functools.partial(
        pl.kernel,
        out_type=jax.ShapeDtypeStruct((BATCH, OUT_DIM), jnp.float32),
        mesh=mesh,
        compiler_params=pltpu.CompilerParams(needs_layout_passes=False),
        scratch_types=[
            pltpu.VMEM((OUT_DIM,), jnp.int32),
            pltpu.VMEM((4, OUT_DIM), jnp.float32),
            pltpu.VMEM((RG * IN_DIM,), jnp.float32),
            pltpu.VMEM((RG, OUT_DIM), jnp.float32),
        ],
    )
    def body(x_hbm, pidx_hbm, coef_hbm, out_hbm, pidx_v, coef_v, x_v, out_v):
        wid = lax.axis_index("s") * NC + lax.axis_index("c")
        row0 = wid * ROWS_PER_W
        pltpu.sync_copy(pidx_hbm, pidx_v)
        pltpu.sync_copy(coef_hbm, coef_v)

        def group(g, carry):
            rbase = row0 + g * RG
            for r in range(RG):
                pltpu.sync_copy(x_hbm.at[rbase + r],
                                x_v.at[pl.ds(r * IN_DIM, IN_DIM)])

            def chunk(j, carry2):
                c = pl.multiple_of(j * L, L)
                pk = pidx_v[pl.ds(c, L)]
                i0 = pk & 0xFFFF
                i1 = lax.shift_right_logical(pk, 16)
                al = coef_v[0, pl.ds(c, L)]
                be = coef_v[1, pl.ds(c, L)]
                ga = coef_v[2, pl.ds(c, L)]
                de = coef_v[3, pl.ds(c, L)]
                for r in range(RG):
                    a = plsc.load_gather(x_v, [i0 + (r * IN_DIM)])
                    b = plsc.load_gather(x_v, [i1 + (r * IN_DIM)])
                    out_v[r, pl.ds(c, L)] = al + be * a + ga * b + de * (a * b)
                return carry2

            lax.fori_loop(0, OUT_DIM // L, chunk, 0)
            pltpu.sync_copy(out_v, out_hbm.at[pl.ds(rbase, RG)])
            return carry

        lax.fori_loop(0, ROWS_PER_W // RG, group, 0)

    return body(x, pidx, coef)


def kernel(x, weight, indices):
    coef, pidx = _coefs(weight.T, indices)
    return _sc_gate_kernel(x, pidx.reshape(OUT_DIM), coef)


# speedup vs baseline: 1.9892x; 1.9892x over previous
import functools

import jax
import jax.numpy as jnp
import numpy as np
from jax import lax
from jax.experimental import pallas as pl
from jax.experimental.pallas import tpu as pltpu
from jax.experimental.pallas import tpu_sc as plsc

IN_DIM = 2048
OUT_DIM = 8192
BATCH = 2048
TAU = 1.0

NC = 2
NS = 16
L = 16
NW = NC * NS
ROWS_PER_W = BATCH // NW
RG = 8

_M = np.zeros((16, 4), np.float32)
for _k, (_al, _be, _ga, _de) in {
    1: (0, 0, 0, 1), 2: (0, 1, 0, -1), 3: (0, 1, 0, 0), 4: (0, 0, 1, -1),
    5: (0, 0, 1, 0), 6: (0, 1, 1, -2), 7: (0, 1, 1, -1), 8: (1, -1, -1, 1),
    9: (1, -1, -1, 2), 10: (1, 0, -1, 0), 11: (1, 0, -1, 1), 12: (1, -1, 0, 0),
    13: (1, -1, 0, 1), 14: (1, 0, 0, -1), 15: (1, 0, 0, 0),
}.items():
    _M[_k] = [_al, _be, _ga, _de]
_MT = _M.T.copy()


def _coef_body(mt_ref, wt_ref, idx_ref, coef_ref, pidx_ref):
    w = wt_ref[...] * (1.0 / TAU)
    m = jnp.max(w, axis=0, keepdims=True)
    e = jnp.exp(w - m)
    p = e / jnp.sum(e, axis=0, keepdims=True)
    coef_ref[...] = jax.lax.dot_general(
        mt_ref[...], p, (((1,), (0,)), ((), ())),
        preferred_element_type=jnp.float32)
    pidx_ref[...] = idx_ref[0:1] | (idx_ref[1:2] << 16)


def _coefs(weight_t, indices):
    return pl.pallas_call(
        _coef_body,
        out_shape=(
            jax.ShapeDtypeStruct((4, OUT_DIM), jnp.float32),
            jax.ShapeDtypeStruct((1, OUT_DIM), jnp.int32),
        ),
    )(jnp.asarray(_MT), weight_t, indices)


def _sc_gate_kernel(x, pidx, coef):
    mesh = plsc.VectorSubcoreMesh(core_axis_name="c", subcore_axis_name="s")

    @functools.partial(
        pl.kernel,
        out_type=jax.ShapeDtypeStruct((BATCH, OUT_DIM), jnp.float32),
        mesh=mesh,
        compiler_params=pltpu.CompilerParams(needs_layout_passes=False),
        scratch_types=[
            pltpu.VMEM((OUT_DIM,), jnp.int32),
            pltpu.VMEM((4, OUT_DIM), jnp.float32),
            pltpu.VMEM((RG * IN_DIM,), jnp.float32),
            pltpu.VMEM((RG, OUT_DIM), jnp.float32),
        ],
    )
    def body(x_hbm, pidx_hbm, coef_hbm, out_hbm, pidx_v, coef_v, x_v, out_v):
        wid = lax.axis_index("s") * NC + lax.axis_index("c")
        row0 = wid * ROWS_PER_W
        pltpu.sync_copy(pidx_hbm, pidx_v)
        pltpu.sync_copy(coef_hbm, coef_v)

        def group(g, carry):
            rbase = row0 + g * RG
            for r in range(RG):
                pltpu.sync_copy(x_hbm.at[rbase + r],
                                x_v.at[pl.ds(r * IN_DIM, IN_DIM)])

            @plsc.parallel_loop(0, OUT_DIM, step=L, unroll=8)
            def chunk(ci):
                c = pl.multiple_of(ci, L)
                pk = pidx_v[pl.ds(c, L)]
                i0 = pk & 0xFFFF
                i1 = lax.shift_right_logical(pk, 16)
                al = coef_v[0, pl.ds(c, L)]
                be = coef_v[1, pl.ds(c, L)]
                ga = coef_v[2, pl.ds(c, L)]
                de = coef_v[3, pl.ds(c, L)]
                for r in range(RG):
                    a = plsc.load_gather(x_v, [i0 + (r * IN_DIM)])
                    b = plsc.load_gather(x_v, [i1 + (r * IN_DIM)])
                    out_v[r, pl.ds(c, L)] = al + be * a + ga * b + de * (a * b)
            pltpu.sync_copy(out_v, out_hbm.at[pl.ds(rbase, RG)])
            return carry

        lax.fori_loop(0, ROWS_PER_W // RG, group, 0)

    return body(x, pidx, coef)


def kernel(x, weight, indices):
    coef, pidx = _coefs(weight.T, indices)
    return _sc_gate_kernel(x, pidx.reshape(OUT_DIM), coef)


# speedup vs baseline: 2.4146x; 1.2139x over previous
import functools

import jax
import jax.numpy as jnp
import numpy as np
from jax import lax
from jax.experimental import pallas as pl
from jax.experimental.pallas import tpu as pltpu
from jax.experimental.pallas import tpu_sc as plsc

IN_DIM = 2048
OUT_DIM = 8192
BATCH = 2048
TAU = 1.0

NC = 2
NS = 16
L = 16
NW = NC * NS
ROWS_PER_W = BATCH // NW
RG = 8
NG = ROWS_PER_W // RG
NQ = 4
QW = OUT_DIM // NQ

_M = np.zeros((16, 4), np.float32)
for _k, (_al, _be, _ga, _de) in {
    1: (0, 0, 0, 1), 2: (0, 1, 0, -1), 3: (0, 1, 0, 0), 4: (0, 0, 1, -1),
    5: (0, 0, 1, 0), 6: (0, 1, 1, -2), 7: (0, 1, 1, -1), 8: (1, -1, -1, 1),
    9: (1, -1, -1, 2), 10: (1, 0, -1, 0), 11: (1, 0, -1, 1), 12: (1, -1, 0, 0),
    13: (1, -1, 0, 1), 14: (1, 0, 0, -1), 15: (1, 0, 0, 0),
}.items():
    _M[_k] = [_al, _be, _ga, _de]
_MT = _M.T.copy()


def _coef_body(mt_ref, wt_ref, idx_ref, coef_ref, pidx_ref):
    w = wt_ref[...] * (1.0 / TAU)
    m = jnp.max(w, axis=0, keepdims=True)
    e = jnp.exp(w - m)
    p = e / jnp.sum(e, axis=0, keepdims=True)
    coef_ref[...] = jax.lax.dot_general(
        mt_ref[...], p, (((1,), (0,)), ((), ())),
        preferred_element_type=jnp.float32)
    pidx_ref[...] = idx_ref[0:1] | (idx_ref[1:2] << 16)


def _coefs(weight_t, indices):
    return pl.pallas_call(
        _coef_body,
        out_shape=(
            jax.ShapeDtypeStruct((4, OUT_DIM), jnp.float32),
            jax.ShapeDtypeStruct((1, OUT_DIM), jnp.int32),
        ),
    )(jnp.asarray(_MT), weight_t, indices)


def _sc_gate_kernel(x_flat, pidx, coef):
    mesh = plsc.VectorSubcoreMesh(core_axis_name="c", subcore_axis_name="s")

    @functools.partial(
        pl.kernel,
        out_type=jax.ShapeDtypeStruct((BATCH, OUT_DIM), jnp.float32),
        mesh=mesh,
        compiler_params=pltpu.CompilerParams(needs_layout_passes=False),
        scratch_types=[
            pltpu.VMEM((OUT_DIM,), jnp.int32),
            pltpu.VMEM((4, OUT_DIM), jnp.float32),
            pltpu.VMEM((RG * IN_DIM,), jnp.float32),
            pltpu.VMEM((RG * IN_DIM,), jnp.float32),
            pltpu.VMEM((RG, QW), jnp.float32),
            pltpu.VMEM((RG, QW), jnp.float32),
            pltpu.SemaphoreType.DMA,
            pltpu.SemaphoreType.DMA,
            pltpu.SemaphoreType.DMA,
            pltpu.SemaphoreType.DMA,
        ],
    )
    def body(x_hbm, pidx_hbm, coef_hbm, out_hbm,
             pidx_v, coef_v, x0, x1, o0, o1, sx0, sx1, so0, so1):
        wid = lax.axis_index("s") * NC + lax.axis_index("c")
        row0 = wid * ROWS_PER_W
        xb = (x0, x1)
        ob = (o0, o1)
        sx = (sx0, sx1)
        so = (so0, so1)

        def x_src(g):
            return x_hbm.at[pl.ds((row0 + g * RG) * IN_DIM, RG * IN_DIM)]

        pltpu.async_copy(x_src(0), x0, sx0)
        pltpu.sync_copy(pidx_hbm, pidx_v)
        pltpu.sync_copy(coef_hbm, coef_v)

        def run_quarter(x_v, o_v, q):
            qc = q * QW

            @plsc.parallel_loop(0, QW, step=L, unroll=8)
            def chunk(ci):
                c = pl.multiple_of(ci, L)
                pk = pidx_v[pl.ds(c + qc, L)]
                i0 = pk & 0xFFFF
                i1 = lax.shift_right_logical(pk, 16)
                al = coef_v[0, pl.ds(c + qc, L)]
                be = coef_v[1, pl.ds(c + qc, L)]
                ga = coef_v[2, pl.ds(c + qc, L)]
                de = coef_v[3, pl.ds(c + qc, L)]
                for r in range(RG):
                    a = plsc.load_gather(x_v, [i0 + (r * IN_DIM)])
                    b = plsc.load_gather(x_v, [i1 + (r * IN_DIM)])
                    o_v[r, pl.ds(c, L)] = al + be * a + ga * b + de * (a * b)

        def out_dst(g, q):
            return out_hbm.at[pl.ds(row0 + g * RG, RG),
                              pl.ds(q * QW, QW)]

        def two_groups(s, carry):
            for b in range(2):
                g = s * 2 + b
                pltpu.make_async_copy(x_src(g), xb[b], sx[b]).wait()

                @pl.when(g + 1 < NG)
                def _():
                    pltpu.async_copy(x_src(g + 1), xb[1 - b], sx[1 - b])

                for q in range(NQ):
                    if q < 2:
                        @pl.when(g >= 1)
                        def _():
                            pltpu.make_async_copy(
                                ob[q % 2], out_dst(g - 1, q + 2),
                                so[q % 2]).wait()
                    else:
                        pltpu.make_async_copy(
                            ob[q % 2], out_dst(g, q - 2), so[q % 2]).wait()
                    run_quarter(xb[b], ob[q % 2], q)
                    pltpu.async_copy(ob[q % 2], out_dst(g, q), so[q % 2])
            return carry

        lax.fori_loop(0, NG // 2, two_groups, 0)
        for q in range(NQ - 2, NQ):
            pltpu.make_async_copy(ob[q % 2], out_dst(NG - 1, q),
                                  so[q % 2]).wait()

    return body(x_flat, pidx, coef)


def kernel(x, weight, indices):
    coef, pidx = _coefs(weight.T, indices)
    return _sc_gate_kernel(x.reshape(BATCH * IN_DIM), pidx.reshape(OUT_DIM),
                           coef)
